# Initial kernel scaffold; baseline (speedup 1.0000x reference)
import jax
import jax.numpy as jnp
from jax.experimental import pallas as pl


def kernel(x, edge_index, edge_weight, batch, conv_W, conv_b, jk_W, jk_b, bn_gamma, bn_beta, lin1_W, lin1_b, lin2_W, lin2_b):
    raise NotImplementedError("write your pallas kernel here")


# speedup vs baseline: 11.3738x; 11.3738x over previous
import functools
import jax
import jax.numpy as jnp
from jax import lax
from jax.experimental import pallas as pl
from jax.experimental.pallas import tpu as pltpu
from jax.experimental.pallas import tpu_sc as plsc

N = 10000
E = 320000
D = 128
B = 64
C = 16
L = 3
EPS = 1e-5

NC = 2
NS = 16
NW = NC * NS
CH = 80
NCH = 128
SEG = 2
SCH = NCH // SEG
EP = NW * NCH * CH
EPT = E // NS
NCH2 = EPT // CH
RPT = 624
TAIL0 = NS * RPT
ZR = 208

BM = 1000
G = N // BM

_mesh = plsc.VectorSubcoreMesh(core_axis_name="c", subcore_axis_name="s",
                               num_cores=NC, num_subcores=NS)

F32 = jnp.float32
HIGH = jax.lax.Precision.HIGHEST


def _zero16():
    return jnp.zeros((16,), F32)


def _sc_deg(c2, ew2):
    def body(c2_hbm, ew2_hbm, degb_hbm, dst_i, val_i, dz, db, bb, deg_sh, dsem):
        ci = lax.axis_index("c")
        si = lax.axis_index("s")

        @pl.when(ci == 0)
        def _():
            pltpu.sync_copy(c2_hbm.at[si], dst_i)
            pltpu.sync_copy(ew2_hbm.at[si], val_i)

            @pl.when(si == 0)
            def _():
                def zrow(i, carry):
                    dz[pl.ds(i * 16, 16)] = _zero16()
                    return carry
                lax.fori_loop(0, 125, zrow, 0)
                for t in range(5):
                    pltpu.sync_copy(dz, deg_sh.at[pl.ds(t * 2000, 2000)])
            plsc.subcore_barrier()

            def grp(g, carry):
                for k in range(8):
                    j = g * 8 + k
                    pltpu.async_copy(val_i.at[j], deg_sh.at[dst_i.at[j]],
                                     dsem, add=True)
                for k in range(8):
                    j = g * 8 + k
                    pltpu.make_async_copy(val_i.at[j],
                                          deg_sh.at[dst_i.at[j]], dsem).wait()
                return carry
            lax.fori_loop(0, NCH2 // 8, grp, 0)
            for j in range(NCH2 - (NCH2 // 8) * 8):
                pltpu.sync_copy(val_i.at[NCH2 - 1 - j],
                                deg_sh.at[dst_i.at[NCH2 - 1 - j]], add=True)
            plsc.subcore_barrier()

            row0 = si * RPT
            pltpu.sync_copy(deg_sh.at[pl.ds(row0, RPT)], db.at[pl.ds(0, RPT)])

            def bc_chunk(t0, nrows, src_off, dst_off):
                def brow(i, carry):
                    w16 = plsc.load_gather(
                        db, [jnp.full((16,), src_off + i, jnp.int32)])
                    for s in range(8):
                        bb[i, pl.ds(s * 16, 16)] = w16
                    return carry
                lax.fori_loop(0, nrows, brow, 0)
                pltpu.sync_copy(bb.at[pl.ds(0, nrows)],
                                degb_hbm.at[pl.ds(dst_off, nrows)])

            for t in range(RPT // ZR):
                bc_chunk(t, ZR, t * ZR, row0 + t * ZR)

            @pl.when(si == NS - 1)
            def _():
                pltpu.sync_copy(deg_sh.at[pl.ds(TAIL0, N - TAIL0)],
                                db.at[pl.ds(0, N - TAIL0)])

                def brow(i, carry):
                    w16 = plsc.load_gather(
                        db, [jnp.full((16,), i, jnp.int32)])
                    for s in range(8):
                        bb[i, pl.ds(s * 16, 16)] = w16
                    return carry
                lax.fori_loop(0, N - TAIL0, brow, 0)
                pltpu.sync_copy(bb.at[pl.ds(0, N - TAIL0)],
                                degb_hbm.at[pl.ds(TAIL0, N - TAIL0)])

    f = pl.kernel(
        body,
        out_type=jax.ShapeDtypeStruct((N, D), F32),
        mesh=_mesh,
        compiler_params=pltpu.CompilerParams(needs_layout_passes=False),
        scratch_types=[
            pltpu.VMEM((NCH2, CH), jnp.int32),
            pltpu.VMEM((NCH2, CH), F32),
            pltpu.VMEM((2000,), F32),
            pltpu.VMEM((RPT,), F32),
            pltpu.VMEM((ZR, D), F32),
            pltpu.VMEM_SHARED((N,), F32),
            pltpu.SemaphoreType.DMA,
        ],
    )
    return f(c2, ew2)


def _scale_rows(rows, ew_i, j):
    def body(j2, carry):
        w16 = plsc.load_gather(
            ew_i, [jnp.full((16,), j, jnp.int32),
                   jnp.full((16,), j2, jnp.int32)])
        for s in range(8):
            rows[j2, pl.ds(s * 16, 16)] = rows[j2, pl.ds(s * 16, 16)] * w16
        return carry
    lax.fori_loop(0, CH, body, 0)


def _prop_body(hw_hbm, r3_hbm, c3_hbm, ew3_hbm, acc_hbm,
               src_i, dst_i, ew_i, rows0, rows1, acc_sh,
               gsem0, gsem1, ssem0, ssem1):
    ci = lax.axis_index("c")
    si = lax.axis_index("s")
    wid = ci * NS + si

    def zrow(i, carry):
        for s in range(8):
            rows0[i, pl.ds(s * 16, 16)] = _zero16()
        return carry
    lax.fori_loop(0, CH, zrow, 0)
    row0 = si * RPT
    for t in range(RPT // CH):
        pltpu.sync_copy(rows0, acc_sh.at[pl.ds(row0 + t * CH, CH)])
    pltpu.sync_copy(rows0.at[pl.ds(0, RPT - (RPT // CH) * CH)],
                    acc_sh.at[pl.ds(row0 + (RPT // CH) * CH,
                                    RPT - (RPT // CH) * CH)])

    @pl.when(si == NS - 1)
    def _():
        pltpu.sync_copy(rows0.at[pl.ds(0, N - TAIL0)],
                        acc_sh.at[pl.ds(TAIL0, N - TAIL0)])
    plsc.subcore_barrier()

    for seg in range(SEG):
        pltpu.sync_copy(r3_hbm.at[wid, pl.ds(seg * SCH, SCH)], src_i)
        pltpu.sync_copy(c3_hbm.at[wid, pl.ds(seg * SCH, SCH)], dst_i)
        pltpu.sync_copy(ew3_hbm.at[wid, pl.ds(seg * SCH, SCH)], ew_i)

        def pair(jj, carry):
            j0 = jj * 2
            j1 = j0 + 1
            d0 = pltpu.async_copy(hw_hbm.at[src_i.at[j0]], rows0, gsem0)
            d1 = pltpu.async_copy(hw_hbm.at[src_i.at[j1]], rows1, gsem1)
            d0.wait()
            _scale_rows(rows0, ew_i, j0)
            s0 = pltpu.async_copy(rows0, acc_sh.at[dst_i.at[j0]], ssem0,
                                  add=True)
            d1.wait()
            _scale_rows(rows1, ew_i, j1)
            s1 = pltpu.async_copy(rows1, acc_sh.at[dst_i.at[j1]], ssem1,
                                  add=True)
            s0.wait()
            s1.wait()
            return carry
        lax.fori_loop(0, SCH // 2, pair, 0)

    plsc.subcore_barrier()
    pltpu.sync_copy(acc_sh.at[pl.ds(row0, RPT)],
                    acc_hbm.at[ci, pl.ds(row0, RPT)])

    @pl.when(si == NS - 1)
    def _():
        pltpu.sync_copy(acc_sh.at[pl.ds(TAIL0, N - TAIL0)],
                        acc_hbm.at[ci, pl.ds(TAIL0, N - TAIL0)])


def _sc_prop(hwp, r3, c3, ew3):
    f = pl.kernel(
        _prop_body,
        out_type=jax.ShapeDtypeStruct((NC, N, D), F32),
        mesh=_mesh,
        compiler_params=pltpu.CompilerParams(needs_layout_passes=False),
        scratch_types=[
            pltpu.VMEM((SCH, CH), jnp.int32),
            pltpu.VMEM((SCH, CH), jnp.int32),
            pltpu.VMEM((SCH, CH), F32),
            pltpu.VMEM((CH, D), F32),
            pltpu.VMEM((CH, D), F32),
            pltpu.VMEM_SHARED((N, D), F32),
            pltpu.SemaphoreType.DMA,
            pltpu.SemaphoreType.DMA,
            pltpu.SemaphoreType.DMA,
            pltpu.SemaphoreType.DMA,
        ],
    )
    return f(hwp, r3, c3, ew3)


def _mm0_body(x_ref, deg_ref, w_ref, out_ref):
    dis = lax.rsqrt(1.0 + deg_ref[...])
    out_ref[...] = jnp.dot(x_ref[...] * dis, w_ref[...],
                           preferred_element_type=F32, precision=HIGH)


def _tc_mm0(x, degb, w):
    return pl.pallas_call(
        _mm0_body,
        grid=(G,),
        in_specs=[
            pl.BlockSpec((BM, D), lambda g: (g, 0)),
            pl.BlockSpec((BM, D), lambda g: (g, 0)),
            pl.BlockSpec((D, D), lambda g: (0, 0)),
        ],
        out_specs=pl.BlockSpec((BM, D), lambda g: (g, 0)),
        out_shape=jax.ShapeDtypeStruct((N, D), F32),
    )(x, degb, w)


def _mid_body(acc_ref, hwp_ref, deg_ref, b_ref, w_ref, h_ref, out_ref):
    dis = lax.rsqrt(1.0 + deg_ref[...])
    h = jnp.maximum(
        (acc_ref[0] + acc_ref[1] + hwp_ref[...]) * dis + b_ref[...], 0.0)
    h_ref[...] = h
    out_ref[...] = jnp.dot(h * dis, w_ref[...],
                           preferred_element_type=F32, precision=HIGH)


def _tc_mid(acc, hwp, degb, b, w):
    return pl.pallas_call(
        _mid_body,
        grid=(G,),
        in_specs=[
            pl.BlockSpec((NC, BM, D), lambda g: (0, g, 0)),
            pl.BlockSpec((BM, D), lambda g: (g, 0)),
            pl.BlockSpec((BM, D), lambda g: (g, 0)),
            pl.BlockSpec((1, D), lambda g: (0, 0)),
            pl.BlockSpec((D, D), lambda g: (0, 0)),
        ],
        out_specs=[
            pl.BlockSpec((BM, D), lambda g: (g, 0)),
            pl.BlockSpec((BM, D), lambda g: (g, 0)),
        ],
        out_shape=[
            jax.ShapeDtypeStruct((N, D), F32),
            jax.ShapeDtypeStruct((N, D), F32),
        ],
    )(acc, hwp, degb, b, w)


def _jk_body(acc_ref, hwp_ref, deg_ref, b_ref, h1_ref, jka_ref, jkb_ref,
             jkbias_ref, batch_ref, wn_ref, out_ref, pooled_ref):
    g = pl.program_id(0)
    dis = lax.rsqrt(1.0 + deg_ref[...])
    h2 = jnp.maximum(
        (acc_ref[0] + acc_ref[1] + hwp_ref[...]) * dis + b_ref[...], 0.0)
    hb = jnp.maximum(
        jnp.dot(h1_ref[...], jka_ref[...], preferred_element_type=F32,
                precision=HIGH)
        + jnp.dot(h2, jkb_ref[...], preferred_element_type=F32, precision=HIGH)
        + jkbias_ref[...], 0.0)
    if wn_ref is not None:
        out_ref[...] = jnp.dot(hb * dis, wn_ref[...],
                               preferred_element_type=F32, precision=HIGH)
    oh_t = (jnp.broadcast_to(batch_ref[0], (B, BM))
            == lax.broadcasted_iota(jnp.int32, (B, BM), 0)).astype(F32)
    p = jnp.dot(oh_t, hb, preferred_element_type=F32, precision=HIGH)

    @pl.when(g == 0)
    def _():
        pooled_ref[...] = p

    @pl.when(g > 0)
    def _():
        pooled_ref[...] = pooled_ref[...] + p


def _tc_jk(acc, hwp, degb, b, h1, jka, jkb, jkbias, batch3, wn):
    has_next = wn is not None
    body = _jk_body if has_next else (
        lambda a, hw, dg, bb, h1r, ja, jb, jbias, bt, po:
        _jk_body(a, hw, dg, bb, h1r, ja, jb, jbias, bt, None, None, po))
    in_specs = [
        pl.BlockSpec((NC, BM, D), lambda g: (0, g, 0)),
        pl.BlockSpec((BM, D), lambda g: (g, 0)),
        pl.BlockSpec((BM, D), lambda g: (g, 0)),
        pl.BlockSpec((1, D), lambda g: (0, 0)),
        pl.BlockSpec((BM, D), lambda g: (g, 0)),
        pl.BlockSpec((D, D), lambda g: (0, 0)),
        pl.BlockSpec((D, D), lambda g: (0, 0)),
        pl.BlockSpec((1, D), lambda g: (0, 0)),
        pl.BlockSpec((1, 1, BM), lambda g: (g, 0, 0)),
    ]
    args = [acc, hwp, degb, b, h1, jka, jkb, jkbias, batch3]
    if has_next:
        in_specs.append(pl.BlockSpec((D, D), lambda g: (0, 0)))
        args.append(wn)
        out_specs = [
            pl.BlockSpec((BM, D), lambda g: (g, 0)),
            pl.BlockSpec((B, D), lambda g: (0, 0)),
        ]
        out_shape = [
            jax.ShapeDtypeStruct((N, D), F32),
            jax.ShapeDtypeStruct((B, D), F32),
        ]
    else:
        out_specs = [pl.BlockSpec((B, D), lambda g: (0, 0))]
        out_shape = [jax.ShapeDtypeStruct((B, D), F32)]
    return pl.pallas_call(
        body,
        grid=(G,),
        in_specs=in_specs,
        out_specs=out_specs,
        out_shape=out_shape,
    )(*args)


def _head_body(p0_ref, p1_ref, p2_ref, g0_ref, g1_ref, g2_ref,
               be0_ref, be1_ref, be2_ref, a0_ref, a1_ref, a2_ref,
               l1b_ref, w2_ref, l2b_ref, out_ref):
    s = 1.0 / jnp.sqrt(1.0 + EPS)
    t = jnp.zeros((B, D), F32)
    for p_ref, g_ref, be_ref, a_ref in (
            (p0_ref, g0_ref, be0_ref, a0_ref),
            (p1_ref, g1_ref, be1_ref, a1_ref),
            (p2_ref, g2_ref, be2_ref, a2_ref)):
        z = p_ref[...] * (g_ref[...] * s) + be_ref[...]
        t = t + jnp.dot(z, a_ref[...], preferred_element_type=F32,
                        precision=HIGH)
    t = jnp.maximum(t + l1b_ref[...], 0.0)
    o = jnp.dot(t, w2_ref[...], preferred_element_type=F32,
                precision=HIGH) + l2b_ref[...]
    m = jnp.max(o, axis=1, keepdims=True)
    e = o - m
    lse = jnp.log(jnp.sum(jnp.exp(e), axis=1, keepdims=True))
    out_ref[...] = e - lse


def _tc_head(p0, p1, p2, g3, be3, a3, l1b, w2, l2b):
    def full(shape):
        return pl.BlockSpec(shape, lambda: tuple(0 for _ in shape))
    return pl.pallas_call(
        _head_body,
        in_specs=[
            full((B, D)), full((B, D)), full((B, D)),
            full((1, D)), full((1, D)), full((1, D)),
            full((1, D)), full((1, D)), full((1, D)),
            full((D, D)), full((D, D)), full((D, D)),
            full((1, D)), full((D, C)), full((1, C)),
        ],
        out_specs=full((B, C)),
        out_shape=jax.ShapeDtypeStruct((B, C), F32),
    )(p0, p1, p2, g3[0], g3[1], g3[2], be3[0], be3[1], be3[2],
      a3[0], a3[1], a3[2], l1b, w2, l2b)


def kernel(x, edge_index, edge_weight, batch, conv_W, conv_b, jk_W, jk_b,
           bn_gamma, bn_beta, lin1_W, lin1_b, lin2_W, lin2_b):
    r = edge_index[0].astype(jnp.int32)
    c = edge_index[1].astype(jnp.int32)
    ew = edge_weight.astype(F32)
    pad = EP - E
    pad_idx = (jnp.arange(pad, dtype=jnp.int32) * 13) % N
    rp = jnp.concatenate([r, pad_idx])
    cp = jnp.concatenate([c, pad_idx])
    ewp = jnp.concatenate([ew, jnp.zeros((pad,), F32)])
    r3 = rp.reshape(NW, NCH, CH)
    c3 = cp.reshape(NW, NCH, CH)
    ew3 = ewp.reshape(NW, NCH, CH)
    c2 = c.reshape(NS, NCH2, CH)
    ew2 = ew.reshape(NS, NCH2, CH)
    batch3 = batch.astype(jnp.int32).reshape(G, 1, BM)

    degb = _sc_deg(c2, ew2)

    bias = conv_b.reshape(L, 2, 1, D)
    jkbias = jk_b.reshape(L, 1, D)
    g3 = bn_gamma.reshape(L, 1, D)
    be3 = bn_beta.reshape(L, 1, D)
    a3 = lin1_W.reshape(L, D, D)
    l1b = lin1_b.reshape(1, D)
    l2b = lin2_b.reshape(1, C)

    pooled = []
    hwp = _tc_mm0(x, degb, conv_W[0, 0])
    for l in range(L):
        acc = _sc_prop(hwp, r3, c3, ew3)
        h1, hwp = _tc_mid(acc, hwp, degb, bias[l, 0], conv_W[l, 1])
        acc = _sc_prop(hwp, r3, c3, ew3)
        wn = conv_W[l + 1, 0] if l < L - 1 else None
        res = _tc_jk(acc, hwp, degb, bias[l, 1], h1,
                     jk_W[l][:D], jk_W[l][D:], jkbias[l], batch3, wn)
        if l < L - 1:
            hwp, p = res
        else:
            (p,) = res
        pooled.append(p)

    return _tc_head(pooled[0], pooled[1], pooled[2], g3, be3, a3,
                    l1b, lin2_W, l2b)


# speedup vs baseline: 17.6168x; 1.5489x over previous
import jax
import jax.numpy as jnp
from jax import lax
from jax.experimental import pallas as pl
from jax.experimental.pallas import tpu as pltpu
from jax.experimental.pallas import tpu_sc as plsc

N = 10000
E = 320000
D = 128
B = 64
C = 16
L = 3
EPS = 1e-5

NC = 2
NS = 16
NW = NC * NS
CH = 80
NCH = 128
SEG = 8
SCH = NCH // SEG
EP = NW * NCH * CH
EPT = E // NS
NCH2 = EPT // CH
RPT = 624
TAIL0 = NS * RPT
ZR = 208

BM = 2000
G = N // BM

_mesh = plsc.VectorSubcoreMesh(core_axis_name="c", subcore_axis_name="s",
                               num_cores=NC, num_subcores=NS)

F32 = jnp.float32
HIGH = jax.lax.Precision.HIGHEST


def _zero16():
    return jnp.zeros((16,), F32)


def _sc_deg(c2, ew2):
    def body(c2_hbm, ew2_hbm, degb_hbm, dst_i, val_i, dz, db, bb, deg_sh, dsem):
        ci = lax.axis_index("c")
        si = lax.axis_index("s")

        @pl.when(ci == 0)
        def _():
            pltpu.sync_copy(c2_hbm.at[si], dst_i)
            pltpu.sync_copy(ew2_hbm.at[si], val_i)

            @pl.when(si == 0)
            def _():
                def zrow(i, carry):
                    dz[pl.ds(i * 16, 16)] = _zero16()
                    return carry
                lax.fori_loop(0, 125, zrow, 0)
                for t in range(5):
                    pltpu.sync_copy(dz, deg_sh.at[pl.ds(t * 2000, 2000)])
            plsc.subcore_barrier()

            def grp(g, carry):
                for k in range(8):
                    j = g * 8 + k
                    pltpu.async_copy(val_i.at[j], deg_sh.at[dst_i.at[j]],
                                     dsem, add=True)
                for k in range(8):
                    j = g * 8 + k
                    pltpu.make_async_copy(val_i.at[j],
                                          deg_sh.at[dst_i.at[j]], dsem).wait()
                return carry
            lax.fori_loop(0, NCH2 // 8, grp, 0)
            for j in range(NCH2 - (NCH2 // 8) * 8):
                pltpu.sync_copy(val_i.at[NCH2 - 1 - j],
                                deg_sh.at[dst_i.at[NCH2 - 1 - j]], add=True)
            plsc.subcore_barrier()

            row0 = si * RPT
            pltpu.sync_copy(deg_sh.at[pl.ds(row0, RPT)], db.at[pl.ds(0, RPT)])

            def bc_chunk(t0, nrows, src_off, dst_off):
                def brow(i, carry):
                    w16 = plsc.load_gather(
                        db, [jnp.full((16,), src_off + i, jnp.int32)])
                    for s in range(8):
                        bb[i, pl.ds(s * 16, 16)] = w16
                    return carry
                lax.fori_loop(0, nrows, brow, 0)
                pltpu.sync_copy(bb.at[pl.ds(0, nrows)],
                                degb_hbm.at[pl.ds(dst_off, nrows)])

            for t in range(RPT // ZR):
                bc_chunk(t, ZR, t * ZR, row0 + t * ZR)

            @pl.when(si == NS - 1)
            def _():
                pltpu.sync_copy(deg_sh.at[pl.ds(TAIL0, N - TAIL0)],
                                db.at[pl.ds(0, N - TAIL0)])

                def brow(i, carry):
                    w16 = plsc.load_gather(
                        db, [jnp.full((16,), i, jnp.int32)])
                    for s in range(8):
                        bb[i, pl.ds(s * 16, 16)] = w16
                    return carry
                lax.fori_loop(0, N - TAIL0, brow, 0)
                pltpu.sync_copy(bb.at[pl.ds(0, N - TAIL0)],
                                degb_hbm.at[pl.ds(TAIL0, N - TAIL0)])

    f = pl.kernel(
        body,
        out_type=jax.ShapeDtypeStruct((N, D), F32),
        mesh=_mesh,
        compiler_params=pltpu.CompilerParams(needs_layout_passes=False),
        scratch_types=[
            pltpu.VMEM((NCH2, CH), jnp.int32),
            pltpu.VMEM((NCH2, CH), F32),
            pltpu.VMEM((2000,), F32),
            pltpu.VMEM((RPT,), F32),
            pltpu.VMEM((ZR, D), F32),
            pltpu.VMEM_SHARED((N,), F32),
            pltpu.SemaphoreType.DMA,
        ],
    )
    return f(c2, ew2)


def _scale_io(rin, rout, ew_i, j):
    def blk(j2):
        w16 = plsc.load_gather(
            ew_i, [jnp.full((16,), j, jnp.int32),
                   jnp.full((16,), j2, jnp.int32)])
        for s in range(8):
            rout[j2, pl.ds(s * 16, 16)] = rin[j2, pl.ds(s * 16, 16)] * w16
    plsc.parallel_loop(0, CH, 1, unroll=4)(blk)


def _prop_body(hw_hbm, r3_hbm, c3_hbm, ew3_hbm, acc_hbm,
               src_i, dst_i, ew_i, rin0, rin1, rout0, rout1, acc_sh,
               gsem0, gsem1, ssem0, ssem1):
    ci = lax.axis_index("c")
    si = lax.axis_index("s")
    wid = ci * NS + si

    def zrow(i, carry):
        for s in range(8):
            rout0[i, pl.ds(s * 16, 16)] = _zero16()
        return carry
    lax.fori_loop(0, CH, zrow, 0)
    row0 = si * RPT
    for t in range(RPT // CH):
        pltpu.sync_copy(rout0, acc_sh.at[pl.ds(row0 + t * CH, CH)])
    pltpu.sync_copy(rout0.at[pl.ds(0, RPT - (RPT // CH) * CH)],
                    acc_sh.at[pl.ds(row0 + (RPT // CH) * CH,
                                    RPT - (RPT // CH) * CH)])

    @pl.when(si == NS - 1)
    def _():
        pltpu.sync_copy(rout0.at[pl.ds(0, N - TAIL0)],
                        acc_sh.at[pl.ds(TAIL0, N - TAIL0)])
    plsc.subcore_barrier()

    rins = (rin0, rin1)
    routs = (rout0, rout1)
    gsems = (gsem0, gsem1)
    ssems = (ssem0, ssem1)

    def gather(j, b):
        pltpu.async_copy(hw_hbm.at[src_i.at[j]], rins[b], gsems[b])

    def scatter(j, b):
        pltpu.async_copy(routs[b], acc_sh.at[dst_i.at[j]], ssems[b], add=True)

    def wait_g(b):
        pltpu.make_async_copy(hw_hbm.at[src_i.at[0]], rins[b],
                              gsems[b]).wait()

    def wait_s(b):
        pltpu.make_async_copy(routs[b], acc_sh.at[dst_i.at[0]],
                              ssems[b]).wait()

    def chunk(j, b, first):
        wait_g(b)
        if not first:
            wait_s(b)
        _scale_io(rins[b], routs[b], ew_i, j)
        scatter(j, b)
        if first:
            gather(j + 2, b)
        else:

            @pl.when(j + 2 < SCH)
            def _():
                gather(j + 2, b)

    def segment(seg, carry):
        pltpu.sync_copy(r3_hbm.at[wid, pl.ds(seg * SCH, SCH)], src_i)
        pltpu.sync_copy(c3_hbm.at[wid, pl.ds(seg * SCH, SCH)], dst_i)
        pltpu.sync_copy(ew3_hbm.at[wid, pl.ds(seg * SCH, SCH)], ew_i)
        gather(0, 0)
        gather(1, 1)
        chunk(0, 0, True)
        chunk(1, 1, True)

        def grp(jj, c2_):
            jb = 2 + jj * 2
            chunk(jb, 0, False)
            chunk(jb + 1, 1, False)
            return c2_
        lax.fori_loop(0, (SCH - 2) // 2, grp, 0)
        wait_s(0)
        wait_s(1)
        return carry
    lax.fori_loop(0, SEG, segment, 0)

    plsc.subcore_barrier()
    pltpu.sync_copy(acc_sh.at[pl.ds(row0, RPT)],
                    acc_hbm.at[ci, pl.ds(row0, RPT)])

    @pl.when(si == NS - 1)
    def _():
        pltpu.sync_copy(acc_sh.at[pl.ds(TAIL0, N - TAIL0)],
                        acc_hbm.at[ci, pl.ds(TAIL0, N - TAIL0)])


def _sc_prop(hwp, r3, c3, ew3):
    f = pl.kernel(
        _prop_body,
        out_type=jax.ShapeDtypeStruct((NC, N, D), F32),
        mesh=_mesh,
        compiler_params=pltpu.CompilerParams(needs_layout_passes=False),
        scratch_types=[
            pltpu.VMEM((SCH, CH), jnp.int32),
            pltpu.VMEM((SCH, CH), jnp.int32),
            pltpu.VMEM((SCH, CH), F32),
            pltpu.VMEM((CH, D), F32),
            pltpu.VMEM((CH, D), F32),
            pltpu.VMEM((CH, D), F32),
            pltpu.VMEM((CH, D), F32),
            pltpu.VMEM_SHARED((N, D), F32),
            pltpu.SemaphoreType.DMA,
            pltpu.SemaphoreType.DMA,
            pltpu.SemaphoreType.DMA,
            pltpu.SemaphoreType.DMA,
        ],
    )
    return f(hwp, r3, c3, ew3)


def _mmraw_body(x_ref, w_ref, out_ref):
    out_ref[...] = jnp.dot(x_ref[...], w_ref[...],
                           preferred_element_type=F32, precision=HIGH)


def _tc_mmraw(x, w):
    return pl.pallas_call(
        _mmraw_body,
        grid=(G,),
        in_specs=[
            pl.BlockSpec((BM, D), lambda g: (g, 0)),
            pl.BlockSpec((D, D), lambda g: (0, 0)),
        ],
        out_specs=pl.BlockSpec((BM, D), lambda g: (g, 0)),
        out_shape=jax.ShapeDtypeStruct((N, D), F32),
    )(x, w)


def _scale0_body(raw_ref, deg_ref, out_ref):
    dis = lax.rsqrt(1.0 + deg_ref[...])
    out_ref[...] = raw_ref[...] * dis


def _tc_scale0(raw, degb):
    return pl.pallas_call(
        _scale0_body,
        grid=(G,),
        in_specs=[
            pl.BlockSpec((BM, D), lambda g: (g, 0)),
            pl.BlockSpec((BM, D), lambda g: (g, 0)),
        ],
        out_specs=pl.BlockSpec((BM, D), lambda g: (g, 0)),
        out_shape=jax.ShapeDtypeStruct((N, D), F32),
    )(raw, degb)


def _mid_body(acc_ref, hwp_ref, deg_ref, b_ref, w_ref, h_ref, out_ref):
    dis = lax.rsqrt(1.0 + deg_ref[...])
    h = jnp.maximum(
        (acc_ref[0] + acc_ref[1] + hwp_ref[...]) * dis + b_ref[...], 0.0)
    h_ref[...] = h
    out_ref[...] = jnp.dot(h * dis, w_ref[...],
                           preferred_element_type=F32, precision=HIGH)


def _tc_mid(acc, hwp, degb, b, w):
    return pl.pallas_call(
        _mid_body,
        grid=(G,),
        in_specs=[
            pl.BlockSpec((NC, BM, D), lambda g: (0, g, 0)),
            pl.BlockSpec((BM, D), lambda g: (g, 0)),
            pl.BlockSpec((BM, D), lambda g: (g, 0)),
            pl.BlockSpec((1, D), lambda g: (0, 0)),
            pl.BlockSpec((D, D), lambda g: (0, 0)),
        ],
        out_specs=[
            pl.BlockSpec((BM, D), lambda g: (g, 0)),
            pl.BlockSpec((BM, D), lambda g: (g, 0)),
        ],
        out_shape=[
            jax.ShapeDtypeStruct((N, D), F32),
            jax.ShapeDtypeStruct((N, D), F32),
        ],
    )(acc, hwp, degb, b, w)


def _jkpre_body(h1_ref, jka_ref, jkbias_ref, out_ref):
    out_ref[...] = jnp.dot(h1_ref[...], jka_ref[...],
                           preferred_element_type=F32,
                           precision=HIGH) + jkbias_ref[...]


def _tc_jkpre(h1, jka, jkbias):
    return pl.pallas_call(
        _jkpre_body,
        grid=(G,),
        in_specs=[
            pl.BlockSpec((BM, D), lambda g: (g, 0)),
            pl.BlockSpec((D, D), lambda g: (0, 0)),
            pl.BlockSpec((1, D), lambda g: (0, 0)),
        ],
        out_specs=pl.BlockSpec((BM, D), lambda g: (g, 0)),
        out_shape=jax.ShapeDtypeStruct((N, D), F32),
    )(h1, jka, jkbias)


def _jk_body(acc_ref, hwp_ref, deg_ref, b_ref, t1_ref, jkb_ref,
             wn_ref, out_ref, hb_ref):
    dis = lax.rsqrt(1.0 + deg_ref[...])
    h2 = jnp.maximum(
        (acc_ref[0] + acc_ref[1] + hwp_ref[...]) * dis + b_ref[...], 0.0)
    hb = jnp.maximum(
        t1_ref[...]
        + jnp.dot(h2, jkb_ref[...], preferred_element_type=F32,
                  precision=HIGH), 0.0)
    hb_ref[...] = hb
    if wn_ref is not None:
        out_ref[...] = jnp.dot(hb * dis, wn_ref[...],
                               preferred_element_type=F32, precision=HIGH)


def _tc_jk(acc, hwp, degb, b, t1, jkb, wn):
    has_next = wn is not None
    body = _jk_body if has_next else (
        lambda a, hw, dg, bb, t1r, jb, hbo:
        _jk_body(a, hw, dg, bb, t1r, jb, None, None, hbo))
    in_specs = [
        pl.BlockSpec((NC, BM, D), lambda g: (0, g, 0)),
        pl.BlockSpec((BM, D), lambda g: (g, 0)),
        pl.BlockSpec((BM, D), lambda g: (g, 0)),
        pl.BlockSpec((1, D), lambda g: (0, 0)),
        pl.BlockSpec((BM, D), lambda g: (g, 0)),
        pl.BlockSpec((D, D), lambda g: (0, 0)),
    ]
    args = [acc, hwp, degb, b, t1, jkb]
    if has_next:
        in_specs.append(pl.BlockSpec((D, D), lambda g: (0, 0)))
        args.append(wn)
        out_specs = [
            pl.BlockSpec((BM, D), lambda g: (g, 0)),
            pl.BlockSpec((BM, D), lambda g: (g, 0)),
        ]
        out_shape = [
            jax.ShapeDtypeStruct((N, D), F32),
            jax.ShapeDtypeStruct((N, D), F32),
        ]
    else:
        out_specs = [pl.BlockSpec((BM, D), lambda g: (g, 0))]
        out_shape = [jax.ShapeDtypeStruct((N, D), F32)]
    return pl.pallas_call(
        body,
        grid=(G,),
        in_specs=in_specs,
        out_specs=out_specs,
        out_shape=out_shape,
    )(*args)


def _pool_body(hb_ref, batch_ref, pooled_ref):
    g = pl.program_id(0)
    oh_t = (jnp.broadcast_to(batch_ref[0], (B, BM))
            == lax.broadcasted_iota(jnp.int32, (B, BM), 0)).astype(F32)
    p = jnp.dot(oh_t, hb_ref[...], preferred_element_type=F32, precision=HIGH)

    @pl.when(g == 0)
    def _():
        pooled_ref[...] = p

    @pl.when(g > 0)
    def _():
        pooled_ref[...] = pooled_ref[...] + p


def _tc_pool(hb, batch3):
    return pl.pallas_call(
        _pool_body,
        grid=(G,),
        in_specs=[
            pl.BlockSpec((BM, D), lambda g: (g, 0)),
            pl.BlockSpec((1, 1, BM), lambda g: (g, 0, 0)),
        ],
        out_specs=pl.BlockSpec((B, D), lambda g: (0, 0)),
        out_shape=jax.ShapeDtypeStruct((B, D), F32),
    )(hb, batch3)


def _head_body(p0_ref, p1_ref, p2_ref, g0_ref, g1_ref, g2_ref,
               be0_ref, be1_ref, be2_ref, a0_ref, a1_ref, a2_ref,
               l1b_ref, w2_ref, l2b_ref, out_ref):
    s = 1.0 / jnp.sqrt(1.0 + EPS)
    t = jnp.zeros((B, D), F32)
    for p_ref, g_ref, be_ref, a_ref in (
            (p0_ref, g0_ref, be0_ref, a0_ref),
            (p1_ref, g1_ref, be1_ref, a1_ref),
            (p2_ref, g2_ref, be2_ref, a2_ref)):
        z = p_ref[...] * (g_ref[...] * s) + be_ref[...]
        t = t + jnp.dot(z, a_ref[...], preferred_element_type=F32,
                        precision=HIGH)
    t = jnp.maximum(t + l1b_ref[...], 0.0)
    o = jnp.dot(t, w2_ref[...], preferred_element_type=F32,
                precision=HIGH) + l2b_ref[...]
    m = jnp.max(o, axis=1, keepdims=True)
    e = o - m
    lse = jnp.log(jnp.sum(jnp.exp(e), axis=1, keepdims=True))
    out_ref[...] = e - lse


def _tc_head(p0, p1, p2, g3, be3, a3, l1b, w2, l2b):
    def full(shape):
        return pl.BlockSpec(shape, lambda: tuple(0 for _ in shape))
    return pl.pallas_call(
        _head_body,
        in_specs=[
            full((B, D)), full((B, D)), full((B, D)),
            full((1, D)), full((1, D)), full((1, D)),
            full((1, D)), full((1, D)), full((1, D)),
            full((D, D)), full((D, D)), full((D, D)),
            full((1, D)), full((D, C)), full((1, C)),
        ],
        out_specs=full((B, C)),
        out_shape=jax.ShapeDtypeStruct((B, C), F32),
    )(p0, p1, p2, g3[0], g3[1], g3[2], be3[0], be3[1], be3[2],
      a3[0], a3[1], a3[2], l1b, w2, l2b)


def kernel(x, edge_index, edge_weight, batch, conv_W, conv_b, jk_W, jk_b,
           bn_gamma, bn_beta, lin1_W, lin1_b, lin2_W, lin2_b):
    r = edge_index[0].astype(jnp.int32)
    c = edge_index[1].astype(jnp.int32)
    ew = edge_weight.astype(F32)
    pad = EP - E
    pad_idx = (jnp.arange(pad, dtype=jnp.int32) * 13) % N
    rp = jnp.concatenate([r, pad_idx])
    cp = jnp.concatenate([c, pad_idx])
    ewp = jnp.concatenate([ew, jnp.zeros((pad,), F32)])
    r3 = rp.reshape(NW, NCH, CH)
    c3 = cp.reshape(NW, NCH, CH)
    ew3 = ewp.reshape(NW, NCH, CH)
    c2 = c.reshape(NS, NCH2, CH)
    ew2 = ew.reshape(NS, NCH2, CH)
    batch3 = batch.astype(jnp.int32).reshape(G, 1, BM)

    degb = _sc_deg(c2, ew2)

    bias = conv_b.reshape(L, 2, 1, D)
    jkbias = jk_b.reshape(L, 1, D)
    g3 = bn_gamma.reshape(L, 1, D)
    be3 = bn_beta.reshape(L, 1, D)
    a3 = lin1_W.reshape(L, D, D)
    l1b = lin1_b.reshape(1, D)
    l2b = lin2_b.reshape(1, C)

    pooled = []
    raw0 = _tc_mmraw(x, conv_W[0, 0])
    hwp = _tc_scale0(raw0, degb)
    for l in range(L):
        acc = _sc_prop(hwp, r3, c3, ew3)
        h1, hwp = _tc_mid(acc, hwp, degb, bias[l, 0], conv_W[l, 1])
        t1 = _tc_jkpre(h1, jk_W[l][:D], jkbias[l])
        acc = _sc_prop(hwp, r3, c3, ew3)
        wn = conv_W[l + 1, 0] if l < L - 1 else None
        res = _tc_jk(acc, hwp, degb, bias[l, 1], t1, jk_W[l][D:], wn)
        if l < L - 1:
            hwp, hb = res
        else:
            (hb,) = res
        pooled.append(_tc_pool(hb, batch3))

    return _tc_head(pooled[0], pooled[1], pooled[2], g3, be3, a3,
                    l1b, lin2_W, l2b)


# speedup vs baseline: 19.2795x; 1.0944x over previous
import jax
import jax.numpy as jnp
from jax import lax
from jax.experimental import pallas as pl
from jax.experimental.pallas import tpu as pltpu
from jax.experimental.pallas import tpu_sc as plsc

N = 10000
E = 320000
D = 128
B = 64
C = 16
L = 3
EPS = 1e-5

NC = 2
NS = 16
NW = NC * NS
CH = 80
NCH = 128
SEG = 4
SCH = NCH // SEG
EP = NW * NCH * CH
EPT = E // NS
NCH2 = EPT // CH
RPT = 624
TAIL0 = NS * RPT
ZR = 208

BM = 2000
G = N // BM

_mesh = plsc.VectorSubcoreMesh(core_axis_name="c", subcore_axis_name="s",
                               num_cores=NC, num_subcores=NS)

F32 = jnp.float32
HIGH = jax.lax.Precision.HIGHEST


def _zero16():
    return jnp.zeros((16,), F32)


def _sc_deg(c2, ew2):
    def body(c2_hbm, ew2_hbm, degb_hbm, dst_i, val_i, dz, db, bb, deg_sh, dsem):
        ci = lax.axis_index("c")
        si = lax.axis_index("s")

        @pl.when(ci == 0)
        def _():
            pltpu.sync_copy(c2_hbm.at[si], dst_i)
            pltpu.sync_copy(ew2_hbm.at[si], val_i)

            @pl.when(si == 0)
            def _():
                def zrow(i, carry):
                    dz[pl.ds(i * 16, 16)] = _zero16()
                    return carry
                lax.fori_loop(0, 125, zrow, 0)
                for t in range(5):
                    pltpu.sync_copy(dz, deg_sh.at[pl.ds(t * 2000, 2000)])
            plsc.subcore_barrier()

            def grp(g, carry):
                for k in range(8):
                    j = g * 8 + k
                    pltpu.async_copy(val_i.at[j], deg_sh.at[dst_i.at[j]],
                                     dsem, add=True)
                for k in range(8):
                    j = g * 8 + k
                    pltpu.make_async_copy(val_i.at[j],
                                          deg_sh.at[dst_i.at[j]], dsem).wait()
                return carry
            lax.fori_loop(0, NCH2 // 8, grp, 0)
            for j in range(NCH2 - (NCH2 // 8) * 8):
                pltpu.sync_copy(val_i.at[NCH2 - 1 - j],
                                deg_sh.at[dst_i.at[NCH2 - 1 - j]], add=True)
            plsc.subcore_barrier()

            row0 = si * RPT
            pltpu.sync_copy(deg_sh.at[pl.ds(row0, RPT)], db.at[pl.ds(0, RPT)])

            def bc_chunk(t0, nrows, src_off, dst_off):
                def brow(i, carry):
                    w16 = plsc.load_gather(
                        db, [jnp.full((16,), src_off + i, jnp.int32)])
                    for s in range(8):
                        bb[i, pl.ds(s * 16, 16)] = w16
                    return carry
                lax.fori_loop(0, nrows, brow, 0)
                pltpu.sync_copy(bb.at[pl.ds(0, nrows)],
                                degb_hbm.at[pl.ds(dst_off, nrows)])

            for t in range(RPT // ZR):
                bc_chunk(t, ZR, t * ZR, row0 + t * ZR)

            @pl.when(si == NS - 1)
            def _():
                pltpu.sync_copy(deg_sh.at[pl.ds(TAIL0, N - TAIL0)],
                                db.at[pl.ds(0, N - TAIL0)])

                def brow(i, carry):
                    w16 = plsc.load_gather(
                        db, [jnp.full((16,), i, jnp.int32)])
                    for s in range(8):
                        bb[i, pl.ds(s * 16, 16)] = w16
                    return carry
                lax.fori_loop(0, N - TAIL0, brow, 0)
                pltpu.sync_copy(bb.at[pl.ds(0, N - TAIL0)],
                                degb_hbm.at[pl.ds(TAIL0, N - TAIL0)])

    f = pl.kernel(
        body,
        out_type=jax.ShapeDtypeStruct((N, D), F32),
        mesh=_mesh,
        compiler_params=pltpu.CompilerParams(needs_layout_passes=False),
        scratch_types=[
            pltpu.VMEM((NCH2, CH), jnp.int32),
            pltpu.VMEM((NCH2, CH), F32),
            pltpu.VMEM((2000,), F32),
            pltpu.VMEM((RPT,), F32),
            pltpu.VMEM((ZR, D), F32),
            pltpu.VMEM_SHARED((N,), F32),
            pltpu.SemaphoreType.DMA,
        ],
    )
    return f(c2, ew2)


def _scale_io(rows, ew_i, j):
    def blk(j2):
        w16 = plsc.load_gather(
            ew_i, [jnp.full((16,), j, jnp.int32),
                   jnp.full((16,), j2, jnp.int32)])
        for s in range(8):
            rows[j2, pl.ds(s * 16, 16)] = rows[j2, pl.ds(s * 16, 16)] * w16
    plsc.parallel_loop(0, CH, 1, unroll=4)(blk)


def _prop_body(hw_hbm, r3_hbm, c3_hbm, ew3_hbm, acc_hbm,
               src_i, dst_i, ew_i, rows0, rows1, rows2, acc_sh,
               gsem0, gsem1, gsem2, ssem0, ssem1, ssem2):
    ci = lax.axis_index("c")
    si = lax.axis_index("s")
    wid = ci * NS + si

    def zrow(i, carry):
        for s in range(8):
            rows0[i, pl.ds(s * 16, 16)] = _zero16()
        return carry
    lax.fori_loop(0, CH, zrow, 0)
    row0 = si * RPT
    for t in range(RPT // CH):
        pltpu.sync_copy(rows0, acc_sh.at[pl.ds(row0 + t * CH, CH)])
    pltpu.sync_copy(rows0.at[pl.ds(0, RPT - (RPT // CH) * CH)],
                    acc_sh.at[pl.ds(row0 + (RPT // CH) * CH,
                                    RPT - (RPT // CH) * CH)])

    @pl.when(si == NS - 1)
    def _():
        pltpu.sync_copy(rows0.at[pl.ds(0, N - TAIL0)],
                        acc_sh.at[pl.ds(TAIL0, N - TAIL0)])
    plsc.subcore_barrier()

    rows = (rows0, rows1, rows2)
    gsems = (gsem0, gsem1, gsem2)
    ssems = (ssem0, ssem1, ssem2)

    def gather(j, b):
        pltpu.async_copy(hw_hbm.at[src_i.at[j]], rows[b], gsems[b])

    def scatter(j, b):
        pltpu.async_copy(rows[b], acc_sh.at[dst_i.at[j]], ssems[b], add=True)

    def wait_g(b):
        pltpu.make_async_copy(hw_hbm.at[src_i.at[0]], rows[b],
                              gsems[b]).wait()

    def wait_s(b):
        pltpu.make_async_copy(rows[b], acc_sh.at[dst_i.at[0]],
                              ssems[b]).wait()

    def chunk(j, b, prefetch):
        wait_g(b)
        _scale_io(rows[b], ew_i, j)
        scatter(j, b)
        if prefetch == "always":
            wait_s((b + 2) % 3)
            gather(j + 2, (b + 2) % 3)
        elif prefetch == "fresh":
            gather(j + 2, (b + 2) % 3)
        elif prefetch == "cond":

            @pl.when(j + 2 < SCH)
            def _():
                wait_s((b + 2) % 3)
                gather(j + 2, (b + 2) % 3)

    def segment(seg, carry):
        pltpu.sync_copy(r3_hbm.at[wid, pl.ds(seg * SCH, SCH)], src_i)
        pltpu.sync_copy(c3_hbm.at[wid, pl.ds(seg * SCH, SCH)], dst_i)
        pltpu.sync_copy(ew3_hbm.at[wid, pl.ds(seg * SCH, SCH)], ew_i)
        gather(0, 0)
        gather(1, 1)
        chunk(0, 0, "fresh")
        chunk(1, 1, "always")

        def grp(jj, c2_):
            jb = 2 + jj * 3
            chunk(jb, 2, "cond")
            chunk(jb + 1, 0, "cond")
            chunk(jb + 2, 1, "cond")
            return c2_
        lax.fori_loop(0, (SCH - 2) // 3, grp, 0)
        for b in range(3):
            wait_s(b)
        return carry
    lax.fori_loop(0, SEG, segment, 0)

    plsc.subcore_barrier()
    pltpu.sync_copy(acc_sh.at[pl.ds(row0, RPT)],
                    acc_hbm.at[ci, pl.ds(row0, RPT)])

    @pl.when(si == NS - 1)
    def _():
        pltpu.sync_copy(acc_sh.at[pl.ds(TAIL0, N - TAIL0)],
                        acc_hbm.at[ci, pl.ds(TAIL0, N - TAIL0)])


def _sc_prop(hwp, r3, c3, ew3):
    f = pl.kernel(
        _prop_body,
        out_type=jax.ShapeDtypeStruct((NC, N, D), F32),
        mesh=_mesh,
        compiler_params=pltpu.CompilerParams(needs_layout_passes=False),
        scratch_types=[
            pltpu.VMEM((SCH, CH), jnp.int32),
            pltpu.VMEM((SCH, CH), jnp.int32),
            pltpu.VMEM((SCH, CH), F32),
            pltpu.VMEM((CH, D), F32),
            pltpu.VMEM((CH, D), F32),
            pltpu.VMEM((CH, D), F32),
            pltpu.VMEM_SHARED((N, D), F32),
            pltpu.SemaphoreType.DMA,
            pltpu.SemaphoreType.DMA,
            pltpu.SemaphoreType.DMA,
            pltpu.SemaphoreType.DMA,
            pltpu.SemaphoreType.DMA,
            pltpu.SemaphoreType.DMA,
        ],
    )
    return f(hwp, r3, c3, ew3)


def _mmraw_body(x_ref, w_ref, out_ref):
    out_ref[...] = jnp.dot(x_ref[...], w_ref[...],
                           preferred_element_type=F32, precision=HIGH)


def _tc_mmraw(x, w):
    return pl.pallas_call(
        _mmraw_body,
        grid=(G,),
        in_specs=[
            pl.BlockSpec((BM, D), lambda g: (g, 0)),
            pl.BlockSpec((D, D), lambda g: (0, 0)),
        ],
        out_specs=pl.BlockSpec((BM, D), lambda g: (g, 0)),
        out_shape=jax.ShapeDtypeStruct((N, D), F32),
    )(x, w)


def _scale0_body(raw_ref, deg_ref, out_ref):
    dis = lax.rsqrt(1.0 + deg_ref[...])
    out_ref[...] = raw_ref[...] * dis


def _tc_scale0(raw, degb):
    return pl.pallas_call(
        _scale0_body,
        grid=(G,),
        in_specs=[
            pl.BlockSpec((BM, D), lambda g: (g, 0)),
            pl.BlockSpec((BM, D), lambda g: (g, 0)),
        ],
        out_specs=pl.BlockSpec((BM, D), lambda g: (g, 0)),
        out_shape=jax.ShapeDtypeStruct((N, D), F32),
    )(raw, degb)


def _mid_body(acc_ref, hwp_ref, deg_ref, b_ref, w_ref, h_ref, out_ref):
    dis = lax.rsqrt(1.0 + deg_ref[...])
    h = jnp.maximum(
        (acc_ref[0] + acc_ref[1] + hwp_ref[...]) * dis + b_ref[...], 0.0)
    h_ref[...] = h
    out_ref[...] = jnp.dot(h * dis, w_ref[...],
                           preferred_element_type=F32, precision=HIGH)


def _tc_mid(acc, hwp, degb, b, w):
    return pl.pallas_call(
        _mid_body,
        grid=(G,),
        in_specs=[
            pl.BlockSpec((NC, BM, D), lambda g: (0, g, 0)),
            pl.BlockSpec((BM, D), lambda g: (g, 0)),
            pl.BlockSpec((BM, D), lambda g: (g, 0)),
            pl.BlockSpec((1, D), lambda g: (0, 0)),
            pl.BlockSpec((D, D), lambda g: (0, 0)),
        ],
        out_specs=[
            pl.BlockSpec((BM, D), lambda g: (g, 0)),
            pl.BlockSpec((BM, D), lambda g: (g, 0)),
        ],
        out_shape=[
            jax.ShapeDtypeStruct((N, D), F32),
            jax.ShapeDtypeStruct((N, D), F32),
        ],
    )(acc, hwp, degb, b, w)


def _jkpre_body(h1_ref, jka_ref, jkbias_ref, out_ref):
    out_ref[...] = jnp.dot(h1_ref[...], jka_ref[...],
                           preferred_element_type=F32,
                           precision=HIGH) + jkbias_ref[...]


def _tc_jkpre(h1, jka, jkbias):
    return pl.pallas_call(
        _jkpre_body,
        grid=(G,),
        in_specs=[
            pl.BlockSpec((BM, D), lambda g: (g, 0)),
            pl.BlockSpec((D, D), lambda g: (0, 0)),
            pl.BlockSpec((1, D), lambda g: (0, 0)),
        ],
        out_specs=pl.BlockSpec((BM, D), lambda g: (g, 0)),
        out_shape=jax.ShapeDtypeStruct((N, D), F32),
    )(h1, jka, jkbias)


def _jk_body(acc_ref, hwp_ref, deg_ref, b_ref, t1_ref, jkb_ref,
             wn_ref, out_ref, hb_ref):
    dis = lax.rsqrt(1.0 + deg_ref[...])
    h2 = jnp.maximum(
        (acc_ref[0] + acc_ref[1] + hwp_ref[...]) * dis + b_ref[...], 0.0)
    hb = jnp.maximum(
        t1_ref[...]
        + jnp.dot(h2, jkb_ref[...], preferred_element_type=F32,
                  precision=HIGH), 0.0)
    hb_ref[...] = hb
    if wn_ref is not None:
        out_ref[...] = jnp.dot(hb * dis, wn_ref[...],
                               preferred_element_type=F32, precision=HIGH)


def _tc_jk(acc, hwp, degb, b, t1, jkb, wn):
    has_next = wn is not None
    body = _jk_body if has_next else (
        lambda a, hw, dg, bb, t1r, jb, hbo:
        _jk_body(a, hw, dg, bb, t1r, jb, None, None, hbo))
    in_specs = [
        pl.BlockSpec((NC, BM, D), lambda g: (0, g, 0)),
        pl.BlockSpec((BM, D), lambda g: (g, 0)),
        pl.BlockSpec((BM, D), lambda g: (g, 0)),
        pl.BlockSpec((1, D), lambda g: (0, 0)),
        pl.BlockSpec((BM, D), lambda g: (g, 0)),
        pl.BlockSpec((D, D), lambda g: (0, 0)),
    ]
    args = [acc, hwp, degb, b, t1, jkb]
    if has_next:
        in_specs.append(pl.BlockSpec((D, D), lambda g: (0, 0)))
        args.append(wn)
        out_specs = [
            pl.BlockSpec((BM, D), lambda g: (g, 0)),
            pl.BlockSpec((BM, D), lambda g: (g, 0)),
        ]
        out_shape = [
            jax.ShapeDtypeStruct((N, D), F32),
            jax.ShapeDtypeStruct((N, D), F32),
        ]
    else:
        out_specs = [pl.BlockSpec((BM, D), lambda g: (g, 0))]
        out_shape = [jax.ShapeDtypeStruct((N, D), F32)]
    return pl.pallas_call(
        body,
        grid=(G,),
        in_specs=in_specs,
        out_specs=out_specs,
        out_shape=out_shape,
    )(*args)


def _pool_body(hb_ref, batch_ref, pooled_ref):
    g = pl.program_id(0)
    oh_t = (jnp.broadcast_to(batch_ref[0], (B, BM))
            == lax.broadcasted_iota(jnp.int32, (B, BM), 0)).astype(F32)
    p = jnp.dot(oh_t, hb_ref[...], preferred_element_type=F32, precision=HIGH)

    @pl.when(g == 0)
    def _():
        pooled_ref[...] = p

    @pl.when(g > 0)
    def _():
        pooled_ref[...] = pooled_ref[...] + p


def _tc_pool(hb, batch3):
    return pl.pallas_call(
        _pool_body,
        grid=(G,),
        in_specs=[
            pl.BlockSpec((BM, D), lambda g: (g, 0)),
            pl.BlockSpec((1, 1, BM), lambda g: (g, 0, 0)),
        ],
        out_specs=pl.BlockSpec((B, D), lambda g: (0, 0)),
        out_shape=jax.ShapeDtypeStruct((B, D), F32),
    )(hb, batch3)


def _head_body(p0_ref, p1_ref, p2_ref, g0_ref, g1_ref, g2_ref,
               be0_ref, be1_ref, be2_ref, a0_ref, a1_ref, a2_ref,
               l1b_ref, w2_ref, l2b_ref, out_ref):
    s = 1.0 / jnp.sqrt(1.0 + EPS)
    t = jnp.zeros((B, D), F32)
    for p_ref, g_ref, be_ref, a_ref in (
            (p0_ref, g0_ref, be0_ref, a0_ref),
            (p1_ref, g1_ref, be1_ref, a1_ref),
            (p2_ref, g2_ref, be2_ref, a2_ref)):
        z = p_ref[...] * (g_ref[...] * s) + be_ref[...]
        t = t + jnp.dot(z, a_ref[...], preferred_element_type=F32,
                        precision=HIGH)
    t = jnp.maximum(t + l1b_ref[...], 0.0)
    o = jnp.dot(t, w2_ref[...], preferred_element_type=F32,
                precision=HIGH) + l2b_ref[...]
    m = jnp.max(o, axis=1, keepdims=True)
    e = o - m
    lse = jnp.log(jnp.sum(jnp.exp(e), axis=1, keepdims=True))
    out_ref[...] = e - lse


def _tc_head(p0, p1, p2, g3, be3, a3, l1b, w2, l2b):
    def full(shape):
        return pl.BlockSpec(shape, lambda: tuple(0 for _ in shape))
    return pl.pallas_call(
        _head_body,
        in_specs=[
            full((B, D)), full((B, D)), full((B, D)),
            full((1, D)), full((1, D)), full((1, D)),
            full((1, D)), full((1, D)), full((1, D)),
            full((D, D)), full((D, D)), full((D, D)),
            full((1, D)), full((D, C)), full((1, C)),
        ],
        out_specs=full((B, C)),
        out_shape=jax.ShapeDtypeStruct((B, C), F32),
    )(p0, p1, p2, g3[0], g3[1], g3[2], be3[0], be3[1], be3[2],
      a3[0], a3[1], a3[2], l1b, w2, l2b)


def kernel(x, edge_index, edge_weight, batch, conv_W, conv_b, jk_W, jk_b,
           bn_gamma, bn_beta, lin1_W, lin1_b, lin2_W, lin2_b):
    r = edge_index[0].astype(jnp.int32)
    c = edge_index[1].astype(jnp.int32)
    ew = edge_weight.astype(F32)
    pad = EP - E
    pad_idx = (jnp.arange(pad, dtype=jnp.int32) * 13) % N
    rp = jnp.concatenate([r, pad_idx])
    cp = jnp.concatenate([c, pad_idx])
    ewp = jnp.concatenate([ew, jnp.zeros((pad,), F32)])
    r3 = rp.reshape(NW, NCH, CH)
    c3 = cp.reshape(NW, NCH, CH)
    ew3 = ewp.reshape(NW, NCH, CH)
    c2 = c.reshape(NS, NCH2, CH)
    ew2 = ew.reshape(NS, NCH2, CH)
    batch3 = batch.astype(jnp.int32).reshape(G, 1, BM)

    degb = _sc_deg(c2, ew2)

    bias = conv_b.reshape(L, 2, 1, D)
    jkbias = jk_b.reshape(L, 1, D)
    g3 = bn_gamma.reshape(L, 1, D)
    be3 = bn_beta.reshape(L, 1, D)
    a3 = lin1_W.reshape(L, D, D)
    l1b = lin1_b.reshape(1, D)
    l2b = lin2_b.reshape(1, C)

    pooled = []
    raw0 = _tc_mmraw(x, conv_W[0, 0])
    hwp = _tc_scale0(raw0, degb)
    for l in range(L):
        acc = _sc_prop(hwp, r3, c3, ew3)
        h1, hwp = _tc_mid(acc, hwp, degb, bias[l, 0], conv_W[l, 1])
        t1 = _tc_jkpre(h1, jk_W[l][:D], jkbias[l])
        acc = _sc_prop(hwp, r3, c3, ew3)
        wn = conv_W[l + 1, 0] if l < L - 1 else None
        res = _tc_jk(acc, hwp, degb, bias[l, 1], t1, jk_W[l][D:], wn)
        if l < L - 1:
            hwp, hb = res
        else:
            (hb,) = res
        pooled.append(_tc_pool(hb, batch3))

    return _tc_head(pooled[0], pooled[1], pooled[2], g3, be3, a3,
                    l1b, lin2_W, l2b)
